# baseline (speedup 1.0000x reference)
import jax
import jax.numpy as jnp
from jax.experimental import pallas as pl

def _probe(x_ref, out_ref):
    out_ref[...] = x_ref[0:16, 0:64] * 2.0

def kernel(x, length, conv_w, conv_b, bn1_gamma, bn1_beta, fc_w, fc_b,
           bn2_gamma, bn2_beta):
    one = jax.lax.optimization_barrier(jnp.float32(1.0))
    y = x.reshape(8192, 128) * one
    return pl.pallas_call(
        _probe,
        out_shape=jax.ShapeDtypeStruct((16, 64), jnp.float32),
    )(y)


# speedup vs baseline: 2.3553x; 2.3553x over previous
import jax
import jax.numpy as jnp
from jax.experimental import pallas as pl

_EPS = 1e-5
_N = 32768
_B = 16


def _fused_kernel(xt_ref, starts_ref, lens_ref, w2_ref, cb_ref, g1_ref, b1_ref,
                  fcw_ref, fcb_ref, g2_ref, b2_ref, out_ref):
    xt = xt_ref[...]
    lens_f = lens_ref[...].astype(jnp.float32)

    sp = jax.lax.dot_general(
        w2_ref[...], xt, dimension_numbers=(((1,), (0,)), ((), ())),
        preferred_element_type=jnp.float32)
    out1 = sp[0:1, :] + cb_ref[0, 0]

    mu1 = jnp.mean(out1)
    d = out1 - mu1
    var1 = jnp.mean(d * d)
    out1n = d / jnp.sqrt(var1 + _EPS) * g1_ref[0, 0] + b1_ref[0, 0]

    s = out1n * sp[1:2, :]

    m = jnp.max(s)
    e = jnp.exp(s - m)

    lane = jax.lax.broadcasted_iota(jnp.int32, (_B, _N), 1)
    starts_i = starts_ref[...]
    mask = (lane >= starts_i) & (lane < starts_i + lens_ref[...])
    me = jnp.where(mask, e, 0.0)

    denom = jnp.sum(me, axis=1, keepdims=True)
    pooled = jax.lax.dot_general(
        me, xt, dimension_numbers=(((1,), (1,)), ((), ())),
        preferred_element_type=jnp.float32)
    pooled = pooled * (1.0 / (denom * lens_f))

    res = jax.lax.dot_general(
        pooled, fcw_ref[...], dimension_numbers=(((1,), (1,)), ((), ())),
        preferred_element_type=jnp.float32) + fcb_ref[...]

    mu2 = jnp.mean(res, axis=0, keepdims=True)
    var2 = jnp.mean((res - mu2) ** 2, axis=0, keepdims=True)
    out_ref[...] = (res - mu2) / jnp.sqrt(var2 + _EPS) * g2_ref[...] + b2_ref[...]


def kernel(x, length, conv_w, conv_b, bn1_gamma, bn1_beta, fc_w, fc_b,
           bn2_gamma, bn2_beta):
    starts = jnp.concatenate(
        [jnp.zeros((1,), dtype=length.dtype), jnp.cumsum(length)[:-1]])
    w2 = jnp.zeros((8, 32), jnp.float32)
    w2 = w2.at[0, :].set(conv_w[0]).at[1, :].set(1.0 / 32.0)
    return pl.pallas_call(
        _fused_kernel,
        out_shape=jax.ShapeDtypeStruct((_B, 64), jnp.float32),
    )(
        x.T,
        starts.reshape(_B, 1),
        length.reshape(_B, 1),
        w2,
        conv_b.reshape(1, 1),
        bn1_gamma.reshape(1, 1),
        bn1_beta.reshape(1, 1),
        fc_w,
        fc_b.reshape(1, 64),
        bn2_gamma.reshape(1, 64),
        bn2_beta.reshape(1, 64),
    )
